# baseline (speedup 1.0000x reference)
import jax
import jax.numpy as jnp
from jax import lax
from jax.experimental import pallas as pl
from jax.experimental.pallas import tpu as pltpu
from jax.experimental.pallas import tpu_sc as plsc

N = 10000
E = 320000
C = 128
D_EMB = 64
NEG = 0.2
ACCW = 136
NSUB = 16
NCORE = 2
EPW = E // (NSUB * NCORE)
K = 40
NCHUNK = EPW // K
CPB = 25
NBPW = NCHUNK // CPB
NP = 10240
RPT = NP // NSUB
BN = 2000


def _proj_body(x_ref, emb_ref, w_ref, ai_ref, aj_ref, hx_ref, si_ref, sj_ref):
    x = x_ref[...]
    e = emb_ref[...]
    w = w_ref[...]
    h = lax.dot_general(x, w, (((1,), (1,)), ((), ())),
                        preferred_element_type=jnp.float32)
    ai = ai_ref[...]
    aj = aj_ref[...]
    si = h @ ai[0:C] + e @ ai[C:C + D_EMB]
    sj = h @ aj[0:C] + e @ aj[C:C + D_EMB]
    si_ref[...] = si
    sj_ref[...] = sj
    hx_ref[...] = jnp.concatenate(
        [h, sj, jnp.zeros((h.shape[0], ACCW - C - 1), jnp.float32)], axis=1)


def _fin_body(acc_ref, si_ref, sj_ref, hx_ref, b_ref, o_ref):
    a = acc_ref[0] + acc_ref[1]
    num = a[:, 0:C]
    den = a[:, C:C + 1]
    s = si_ref[...] + sj_ref[...]
    exs = jnp.exp(jnp.where(s > 0, s, NEG * s))
    num = num + exs * hx_ref[:, 0:C]
    den = den + exs
    o_ref[...] = jnp.maximum(num / den + b_ref[...], 0.0)


def _sc_body(si_hbm, src_hbm, dst_hbm, hx_hbm, out_hbm,
             si_t, srcb, dstb, grow, acc, gsem0, gsem1, ssem0, ssem1):
    cid = lax.axis_index("c")
    sid = lax.axis_index("s")
    wid = cid * NSUB + sid
    lane = lax.iota(jnp.int32, 16)
    zv = jnp.zeros((16,), jnp.float32)
    gsem = (gsem0, gsem1)
    ssem = (ssem0, ssem1)

    def _zrow(i, carry):
        for j in range(C // 16):
            grow[0, i, pl.ds(j * 16, 16)] = zv
        grow[0, i, pl.ds(ACCW - 16, 16)] = zv
        return carry

    lax.fori_loop(0, K, _zrow, 0)

    def _zcp(k, carry):
        pltpu.sync_copy(grow.at[0], acc.at[pl.ds(sid * RPT + k * K, K), :])
        return carry

    lax.fori_loop(0, RPT // K, _zcp, 0)

    pltpu.sync_copy(si_hbm, si_t)

    plsc.subcore_barrier()

    col_sj = jnp.full((16,), C, jnp.int32)

    def _ldblk(blk):
        pltpu.sync_copy(src_hbm.at[wid * NBPW + blk],
                        srcb.at[pl.ds((blk % 2) * CPB, CPB), :])
        pltpu.sync_copy(dst_hbm.at[wid * NBPW + blk],
                        dstb.at[pl.ds((blk % 2) * CPB, CPB), :])

    def _srow(ci):
        return ((ci // CPB) % 2) * CPB + ci % CPB

    def _issue(b, ci):
        pltpu.async_copy(hx_hbm.at[srcb.at[_srow(ci)]], grow.at[b], gsem[b])

    def _step(b, ci):
        pltpu.make_async_copy(hx_hbm.at[srcb.at[_srow(ci)]], grow.at[b],
                              gsem[b]).wait()
        o = 1 - b

        @pl.when(ci + 1 < NCHUNK)
        def _prefetch():
            @pl.when(ci % CPB == CPB - 1)
            def _nextblk():
                _ldblk((ci + 1) // CPB)

            _issue(o, ci + 1)

        r = _srow(ci)

        def _egrp(g, c2):
            rows16 = g * 16 + lane
            iv = dstb[r, pl.ds(g * 16, 16)]
            si_v = plsc.load_gather(si_t, [iv])
            sj_v = plsc.load_gather(grow.at[b], [rows16, col_sj])
            a = si_v + sj_v
            a = jnp.where(a > 0, a, a * NEG)
            exv = jnp.exp(a)
            for l in range(16):
                e = g * 16 + l
                ex = exv[l]
                for j in range(C // 16):
                    grow[b, e, pl.ds(j * 16, 16)] = (
                        grow[b, e, pl.ds(j * 16, 16)] * ex)
                tv = grow[b, e, pl.ds(ACCW - 16, 16)]
                tail = jnp.where(lane < 8, tv,
                                 jnp.where(lane == 8, ex, 0.0))
                grow[b, e, pl.ds(ACCW - 16, 16)] = tail
            return c2

        lax.fori_loop(0, 0, _egrp, 0)

    _ldblk(0)
    _issue(0, 0)

    def _chunk(ci, carry):
        @pl.when(ci % 2 == 0)
        def _even():
            _step(0, ci)

        @pl.when(ci % 2 == 1)
        def _odd():
            _step(1, ci)

        return carry

    lax.fori_loop(0, NCHUNK, _chunk, 0)


    plsc.subcore_barrier()
    pltpu.sync_copy(acc.at[pl.ds(sid * RPT, RPT), :],
                    out_hbm.at[cid, pl.ds(sid * RPT, RPT), :])


def _proj_call(x, emb, w, ai, aj):
    return pl.pallas_call(
        _proj_body,
        grid=(N // BN,),
        in_specs=[
            pl.BlockSpec((BN, C), lambda i: (i, 0)),
            pl.BlockSpec((BN, D_EMB), lambda i: (i, 0)),
            pl.BlockSpec((C, C), lambda i: (0, 0)),
            pl.BlockSpec((C + D_EMB, 1), lambda i: (0, 0)),
            pl.BlockSpec((C + D_EMB, 1), lambda i: (0, 0)),
        ],
        out_specs=[
            pl.BlockSpec((BN, ACCW), lambda i: (i, 0)),
            pl.BlockSpec((BN, 1), lambda i: (i, 0)),
            pl.BlockSpec((BN, 1), lambda i: (i, 0)),
        ],
        out_shape=[
            jax.ShapeDtypeStruct((N, ACCW), jnp.float32),
            jax.ShapeDtypeStruct((N, 1), jnp.float32),
            jax.ShapeDtypeStruct((N, 1), jnp.float32),
        ],
    )(x, emb, w, ai, aj)


def _fin_call(acc, si, sj, hx, bias):
    return pl.pallas_call(
        _fin_body,
        grid=(N // BN,),
        in_specs=[
            pl.BlockSpec((NCORE, BN, ACCW), lambda i: (0, i, 0)),
            pl.BlockSpec((BN, 1), lambda i: (i, 0)),
            pl.BlockSpec((BN, 1), lambda i: (i, 0)),
            pl.BlockSpec((BN, ACCW), lambda i: (i, 0)),
            pl.BlockSpec((1, C), lambda i: (0, 0)),
        ],
        out_specs=pl.BlockSpec((BN, C), lambda i: (i, 0)),
        out_shape=jax.ShapeDtypeStruct((N, C), jnp.float32),
    )(acc, si, sj, hx, bias)


def _sc_call(si, src, dst, hx):
    mesh = plsc.VectorSubcoreMesh(core_axis_name="c", subcore_axis_name="s")
    f = pl.kernel(
        _sc_body,
        out_type=jax.ShapeDtypeStruct((NCORE, NP, ACCW), jnp.float32),
        mesh=mesh,
        compiler_params=pltpu.CompilerParams(use_tc_tiling_on_sc=False,
                                             needs_layout_passes=False),
        scratch_types=[
            pltpu.VMEM((N,), jnp.float32),
            pltpu.VMEM((2 * CPB, K), jnp.int32),
            pltpu.VMEM((2 * CPB, K), jnp.int32),
            pltpu.VMEM((2, K, ACCW), jnp.float32),
            pltpu.VMEM_SHARED((NP, ACCW), jnp.float32),
            pltpu.SemaphoreType.DMA,
            pltpu.SemaphoreType.DMA,
            pltpu.SemaphoreType.DMA,
            pltpu.SemaphoreType.DMA,
        ],
    )
    return f(si, src, dst, hx)


def kernel(x, edge_index, embedding, W, att_i, att_j, bias):
    hx, si, sj = _proj_call(x, embedding, W,
                            att_i.reshape(-1, 1), att_j.reshape(-1, 1))
    acc = _sc_call(si.reshape(-1),
                   edge_index[0].reshape(-1, CPB, K),
                   edge_index[1].reshape(-1, CPB, K), hx)
    return _fin_call(acc, si, sj, hx, bias.reshape(1, C))


# speedup vs baseline: 1.3320x; 1.3320x over previous
import jax
import jax.numpy as jnp
from jax import lax
from jax.experimental import pallas as pl
from jax.experimental.pallas import tpu as pltpu
from jax.experimental.pallas import tpu_sc as plsc

N = 10000
E = 320000
C = 128
D_EMB = 64
NEG = 0.2
ACCW = 136
NSUB = 16
NCORE = 2
EPW = E // (NSUB * NCORE)
K = 80
NCHUNK = EPW // K
CPB = 5
NBPW = NCHUNK // CPB
NBUF = 3
NP = 10112
RPT = NP // NSUB
BN = 2000


def _proj_body(x_ref, emb_ref, w_ref, ai_ref, aj_ref, hx_ref, si_ref, sj_ref):
    x = x_ref[...]
    e = emb_ref[...]
    w = w_ref[...]
    h = lax.dot_general(x, w, (((1,), (1,)), ((), ())),
                        preferred_element_type=jnp.float32)
    ai = ai_ref[...]
    aj = aj_ref[...]
    si = h @ ai[0:C] + e @ ai[C:C + D_EMB]
    sj = h @ aj[0:C] + e @ aj[C:C + D_EMB]
    si_ref[...] = si
    sj_ref[...] = sj
    hx_ref[...] = jnp.concatenate(
        [h, sj, jnp.zeros((h.shape[0], ACCW - C - 1), jnp.float32)], axis=1)


def _fin_body(acc_ref, si_ref, sj_ref, hx_ref, b_ref, o_ref):
    a = acc_ref[0] + acc_ref[1]
    num = a[:, 0:C]
    den = a[:, C:C + 1]
    s = si_ref[...] + sj_ref[...]
    exs = jnp.exp(jnp.where(s > 0, s, NEG * s))
    num = num + exs * hx_ref[:, 0:C]
    den = den + exs
    o_ref[...] = jnp.maximum(num / den + b_ref[...], 0.0)


def _sc_body(si_hbm, src_hbm, dst_hbm, hx_hbm, out_hbm,
             si_t, srcb, dstb, grow, acc,
             gsem0, gsem1, gsem2, ssem0, ssem1, ssem2):
    cid = lax.axis_index("c")
    sid = lax.axis_index("s")
    wid = cid * NSUB + sid
    lane = lax.iota(jnp.int32, 16)
    zv = jnp.zeros((16,), jnp.float32)
    gsem = (gsem0, gsem1, gsem2)
    ssem = (ssem0, ssem1, ssem2)

    def _zrow(i, carry):
        for j in range(C // 16):
            grow[0, i, pl.ds(j * 16, 16)] = zv
        grow[0, i, pl.ds(ACCW - 16, 16)] = zv
        return carry

    lax.fori_loop(0, K, _zrow, 0)

    def _zcp(k, carry):
        pltpu.sync_copy(grow.at[0], acc.at[pl.ds(sid * RPT + k * K, K), :])
        return carry

    lax.fori_loop(0, RPT // K, _zcp, 0)
    pltpu.sync_copy(grow.at[0, pl.ds(0, RPT - (RPT // K) * K), :],
                    acc.at[pl.ds(sid * RPT + (RPT // K) * K,
                                 RPT - (RPT // K) * K), :])

    pltpu.sync_copy(si_hbm, si_t)

    plsc.subcore_barrier()

    col_sj = jnp.full((16,), C, jnp.int32)

    def _ldblk(blk):
        pltpu.sync_copy(src_hbm.at[wid * NBPW + blk],
                        srcb.at[pl.ds((blk % 2) * CPB, CPB), :])
        pltpu.sync_copy(dst_hbm.at[wid * NBPW + blk],
                        dstb.at[pl.ds((blk % 2) * CPB, CPB), :])

    def _srow(ci):
        return ((ci // CPB) % 2) * CPB + ci % CPB

    def _issue(b, ci):
        pltpu.async_copy(hx_hbm.at[srcb.at[_srow(ci)]], grow.at[b], gsem[b])

    def _step(b, ci):
        o = (b + 2) % NBUF

        @pl.when(ci + 2 < NCHUNK)
        def _prefetch():
            @pl.when((ci + 2) % CPB == 0)
            def _nextblk():
                _ldblk((ci + 2) // CPB)

            @pl.when(ci >= 1)
            def _drain():
                pltpu.make_async_copy(grow.at[o],
                                      acc.at[dstb.at[_srow(ci - 1)]],
                                      ssem[o]).wait()
            _issue(o, ci + 2)

        pltpu.make_async_copy(hx_hbm.at[srcb.at[_srow(ci)]], grow.at[b],
                              gsem[b]).wait()

        r = _srow(ci)

        def _egrp(g, c2):
            rows16 = g * 16 + lane
            iv = dstb[r, pl.ds(g * 16, 16)]
            si_v = plsc.load_gather(si_t, [iv])
            sj_v = plsc.load_gather(grow.at[b], [rows16, col_sj])
            a = si_v + sj_v
            a = jnp.where(a > 0, a, a * NEG)
            exv = jnp.exp(a)
            for l in range(16):
                e = g * 16 + l
                ex = exv[l]
                for j in range(C // 16):
                    grow[b, e, pl.ds(j * 16, 16)] = (
                        grow[b, e, pl.ds(j * 16, 16)] * ex)
                tv = grow[b, e, pl.ds(ACCW - 16, 16)]
                tail = jnp.where(lane < 8, tv,
                                 jnp.where(lane == 8, ex, 0.0))
                grow[b, e, pl.ds(ACCW - 16, 16)] = tail
            return c2

        lax.fori_loop(0, K // 16, _egrp, 0)
        pltpu.async_copy(grow.at[b], acc.at[dstb.at[r]], ssem[b], add=True)

    _ldblk(0)
    _issue(0, 0)
    _issue(1, 1)

    def _chunk(ci, carry):
        @pl.when(ci % NBUF == 0)
        def _b0():
            _step(0, ci)

        @pl.when(ci % NBUF == 1)
        def _b1():
            _step(1, ci)

        @pl.when(ci % NBUF == 2)
        def _b2():
            _step(2, ci)

        return carry

    lax.fori_loop(0, NCHUNK, _chunk, 0)

    pltpu.make_async_copy(grow.at[(NCHUNK - 3) % NBUF],
                          acc.at[dstb.at[_srow(NCHUNK - 3)]],
                          ssem[(NCHUNK - 3) % NBUF]).wait()
    pltpu.make_async_copy(grow.at[(NCHUNK - 2) % NBUF],
                          acc.at[dstb.at[_srow(NCHUNK - 2)]],
                          ssem[(NCHUNK - 2) % NBUF]).wait()
    pltpu.make_async_copy(grow.at[(NCHUNK - 1) % NBUF],
                          acc.at[dstb.at[_srow(NCHUNK - 1)]],
                          ssem[(NCHUNK - 1) % NBUF]).wait()

    plsc.subcore_barrier()
    pltpu.sync_copy(acc.at[pl.ds(sid * RPT, RPT), :],
                    out_hbm.at[cid, pl.ds(sid * RPT, RPT), :])


def _proj_call(x, emb, w, ai, aj):
    return pl.pallas_call(
        _proj_body,
        grid=(N // BN,),
        in_specs=[
            pl.BlockSpec((BN, C), lambda i: (i, 0)),
            pl.BlockSpec((BN, D_EMB), lambda i: (i, 0)),
            pl.BlockSpec((C, C), lambda i: (0, 0)),
            pl.BlockSpec((C + D_EMB, 1), lambda i: (0, 0)),
            pl.BlockSpec((C + D_EMB, 1), lambda i: (0, 0)),
        ],
        out_specs=[
            pl.BlockSpec((BN, ACCW), lambda i: (i, 0)),
            pl.BlockSpec((BN, 1), lambda i: (i, 0)),
            pl.BlockSpec((BN, 1), lambda i: (i, 0)),
        ],
        out_shape=[
            jax.ShapeDtypeStruct((N, ACCW), jnp.float32),
            jax.ShapeDtypeStruct((N, 1), jnp.float32),
            jax.ShapeDtypeStruct((N, 1), jnp.float32),
        ],
    )(x, emb, w, ai, aj)


def _fin_call(acc, si, sj, hx, bias):
    return pl.pallas_call(
        _fin_body,
        grid=(N // BN,),
        in_specs=[
            pl.BlockSpec((NCORE, BN, ACCW), lambda i: (0, i, 0)),
            pl.BlockSpec((BN, 1), lambda i: (i, 0)),
            pl.BlockSpec((BN, 1), lambda i: (i, 0)),
            pl.BlockSpec((BN, ACCW), lambda i: (i, 0)),
            pl.BlockSpec((1, C), lambda i: (0, 0)),
        ],
        out_specs=pl.BlockSpec((BN, C), lambda i: (i, 0)),
        out_shape=jax.ShapeDtypeStruct((N, C), jnp.float32),
    )(acc, si, sj, hx, bias)


def _sc_call(si, src, dst, hx):
    mesh = plsc.VectorSubcoreMesh(core_axis_name="c", subcore_axis_name="s")
    f = pl.kernel(
        _sc_body,
        out_type=jax.ShapeDtypeStruct((NCORE, NP, ACCW), jnp.float32),
        mesh=mesh,
        compiler_params=pltpu.CompilerParams(use_tc_tiling_on_sc=False,
                                             needs_layout_passes=False),
        scratch_types=[
            pltpu.VMEM((N,), jnp.float32),
            pltpu.VMEM((2 * CPB, K), jnp.int32),
            pltpu.VMEM((2 * CPB, K), jnp.int32),
            pltpu.VMEM((NBUF, K, ACCW), jnp.float32),
            pltpu.VMEM_SHARED((NP, ACCW), jnp.float32),
            pltpu.SemaphoreType.DMA,
            pltpu.SemaphoreType.DMA,
            pltpu.SemaphoreType.DMA,
            pltpu.SemaphoreType.DMA,
            pltpu.SemaphoreType.DMA,
            pltpu.SemaphoreType.DMA,
        ],
    )
    return f(si, src, dst, hx)


def kernel(x, edge_index, embedding, W, att_i, att_j, bias):
    hx, si, sj = _proj_call(x, embedding, W,
                            att_i.reshape(-1, 1), att_j.reshape(-1, 1))
    acc = _sc_call(si.reshape(-1),
                   edge_index[0].reshape(-1, CPB, K),
                   edge_index[1].reshape(-1, CPB, K), hx)
    return _fin_call(acc, si, sj, hx, bias.reshape(1, C))
